# baseline (speedup 1.0000x reference)
import functools

import jax
import jax.numpy as jnp
from jax import lax
from jax.experimental import pallas as pl
from jax.experimental.pallas import tpu as pltpu
from jax.experimental.pallas import tpu_sc as plsc

VOCAB = 100000
DIM = 128
BATCH = 1024
CTX = 50

NC = 2
NS = 16
L = 16
NW = NC * NS
ROWS_PER_W = BATCH // NW


def _sc_gather_sum(x, emb_table):
    mesh = plsc.VectorSubcoreMesh(core_axis_name="c", subcore_axis_name="s")

    def _accum(rows_v, acc_v, r):
        for c in range(DIM // L):
            sl = pl.ds(c * L, L)
            vals = [rows_v[rr, sl] for rr in range(CTX)]
            while len(vals) > 1:
                nxt = [a + b for a, b in zip(vals[::2], vals[1::2])]
                if len(vals) % 2:
                    nxt.append(vals[-1])
                vals = nxt
            acc_v[r, sl] = vals[0]

    @functools.partial(
        pl.kernel,
        out_type=jax.ShapeDtypeStruct((BATCH, DIM), jnp.float32),
        mesh=mesh,
        scratch_types=[
            pltpu.VMEM((ROWS_PER_W, CTX), jnp.int32),
            pltpu.VMEM((CTX, DIM), jnp.float32),
            pltpu.VMEM((CTX, DIM), jnp.float32),
            pltpu.VMEM((ROWS_PER_W, DIM), jnp.float32),
            pltpu.SemaphoreType.DMA,
            pltpu.SemaphoreType.DMA,
        ],
    )
    def k(x_hbm, tbl_hbm, out_hbm, idx_v, rows_a, rows_b, acc_v, sem_a, sem_b):
        wid = lax.axis_index("s") * NC + lax.axis_index("c")
        base = wid * ROWS_PER_W
        pltpu.sync_copy(x_hbm.at[pl.ds(base, ROWS_PER_W)], idx_v)

        pltpu.async_copy(tbl_hbm.at[idx_v.at[0]], rows_a, sem_a)
        pltpu.async_copy(tbl_hbm.at[idx_v.at[1]], rows_b, sem_b)

        @pl.loop(0, ROWS_PER_W, step=2)
        def _(r):
            pltpu.make_async_copy(tbl_hbm.at[idx_v.at[r]], rows_a, sem_a).wait()
            _accum(rows_a, acc_v, r)

            @pl.when(r + 2 < ROWS_PER_W)
            def _():
                pltpu.async_copy(tbl_hbm.at[idx_v.at[r + 2]], rows_a, sem_a)

            pltpu.make_async_copy(
                tbl_hbm.at[idx_v.at[r + 1]], rows_b, sem_b
            ).wait()
            _accum(rows_b, acc_v, r + 1)

            @pl.when(r + 3 < ROWS_PER_W)
            def _():
                pltpu.async_copy(tbl_hbm.at[idx_v.at[r + 3]], rows_b, sem_b)

        pltpu.sync_copy(acc_v, out_hbm.at[pl.ds(base, ROWS_PER_W)])

    return k(x, emb_table)


VB = 4096
_GRID = (VOCAB + VB - 1) // VB


def _tc_project_t(h, W, brow):
    def mm(h_ref, w_ref, b_ref, o_ref):
        hb = h_ref[...].astype(jnp.bfloat16)
        wb = w_ref[...]
        acc = lax.dot_general(
            wb, hb, (((1,), (1,)), ((), ())),
            preferred_element_type=jnp.float32,
        )
        o_ref[...] = acc + b_ref[...].T

    return pl.pallas_call(
        mm,
        grid=(_GRID,),
        in_specs=[
            pl.BlockSpec((BATCH, DIM), lambda j: (0, 0)),
            pl.BlockSpec((VB, DIM), lambda j: (j, 0)),
            pl.BlockSpec((1, VB), lambda j: (0, j)),
        ],
        out_specs=pl.BlockSpec((VB, BATCH), lambda j: (j, 0)),
        out_shape=jax.ShapeDtypeStruct((VOCAB, BATCH), jnp.float32),
        compiler_params=pltpu.CompilerParams(
            dimension_semantics=("arbitrary",),
        ),
    )(h, W, brow)


def kernel(x, emb_table, W, b):
    x = x.astype(jnp.int32)
    h = _sc_gather_sum(x, emb_table)
    lt = _tc_project_t(h, W.astype(jnp.bfloat16), b.reshape(1, VOCAB))
    return lt.T


# speedup vs baseline: 1.0286x; 1.0286x over previous
import functools

import jax
import jax.numpy as jnp
from jax import lax
from jax.experimental import pallas as pl
from jax.experimental.pallas import tpu as pltpu
from jax.experimental.pallas import tpu_sc as plsc

VOCAB = 100000
DIM = 128
BATCH = 1024
CTX = 50

NC = 2
NS = 16
L = 16
NW = NC * NS
ROWS_PER_W = BATCH // NW


def _sc_gather_sum(x, emb_table):
    mesh = plsc.VectorSubcoreMesh(core_axis_name="c", subcore_axis_name="s")

    def _accum(rows_v, acc_v, r):
        for c in range(DIM // L):
            sl = pl.ds(c * L, L)
            vals = [rows_v[rr, sl] for rr in range(CTX)]
            while len(vals) > 1:
                nxt = [a + b for a, b in zip(vals[::2], vals[1::2])]
                if len(vals) % 2:
                    nxt.append(vals[-1])
                vals = nxt
            acc_v[r, sl] = vals[0]

    @functools.partial(
        pl.kernel,
        out_type=jax.ShapeDtypeStruct((BATCH, DIM), jnp.float32),
        mesh=mesh,
        scratch_types=[
            pltpu.VMEM((ROWS_PER_W, CTX), jnp.int32),
            pltpu.VMEM((CTX, DIM), jnp.float32),
            pltpu.VMEM((CTX, DIM), jnp.float32),
            pltpu.VMEM((ROWS_PER_W, DIM), jnp.float32),
            pltpu.SemaphoreType.DMA,
            pltpu.SemaphoreType.DMA,
        ],
    )
    def k(x_hbm, tbl_hbm, out_hbm, idx_v, rows_a, rows_b, acc_v, sem_a, sem_b):
        wid = lax.axis_index("s") * NC + lax.axis_index("c")
        base = wid * ROWS_PER_W
        pltpu.sync_copy(x_hbm.at[pl.ds(base, ROWS_PER_W)], idx_v)

        pltpu.async_copy(tbl_hbm.at[idx_v.at[0]], rows_a, sem_a)
        pltpu.async_copy(tbl_hbm.at[idx_v.at[1]], rows_b, sem_b)

        @pl.loop(0, ROWS_PER_W, step=2)
        def _(r):
            pltpu.make_async_copy(tbl_hbm.at[idx_v.at[r]], rows_a, sem_a).wait()
            _accum(rows_a, acc_v, r)

            @pl.when(r + 2 < ROWS_PER_W)
            def _():
                pltpu.async_copy(tbl_hbm.at[idx_v.at[r + 2]], rows_a, sem_a)

            pltpu.make_async_copy(
                tbl_hbm.at[idx_v.at[r + 1]], rows_b, sem_b
            ).wait()
            _accum(rows_b, acc_v, r + 1)

            @pl.when(r + 3 < ROWS_PER_W)
            def _():
                pltpu.async_copy(tbl_hbm.at[idx_v.at[r + 3]], rows_b, sem_b)

        pltpu.sync_copy(acc_v, out_hbm.at[pl.ds(base, ROWS_PER_W)])

    return k(x, emb_table)


VB = 5120
_GRID = (VOCAB + VB - 1) // VB


def _tc_project_t(h, W, brow):
    def mm(h_ref, w_ref, b_ref, o_ref):
        hb = h_ref[...].astype(jnp.bfloat16)
        wb = w_ref[...].astype(jnp.bfloat16)
        acc = lax.dot_general(
            wb, hb, (((1,), (1,)), ((), ())),
            preferred_element_type=jnp.float32,
        )
        o_ref[...] = acc + b_ref[...].T

    return pl.pallas_call(
        mm,
        grid=(_GRID,),
        in_specs=[
            pl.BlockSpec((BATCH, DIM), lambda j: (0, 0)),
            pl.BlockSpec((VB, DIM), lambda j: (j, 0)),
            pl.BlockSpec((1, VB), lambda j: (0, j)),
        ],
        out_specs=pl.BlockSpec((VB, BATCH), lambda j: (j, 0)),
        out_shape=jax.ShapeDtypeStruct((VOCAB, BATCH), jnp.float32),
        compiler_params=pltpu.CompilerParams(
            dimension_semantics=("arbitrary",),
        ),
    )(h, W, brow)


def kernel(x, emb_table, W, b):
    x = x.astype(jnp.int32)
    h = _sc_gather_sum(x, emb_table)
    lt = _tc_project_t(h, W, b.reshape(1, VOCAB))
    return lt.T


# speedup vs baseline: 1.0319x; 1.0032x over previous
import functools

import jax
import jax.numpy as jnp
from jax import lax
from jax.experimental import pallas as pl
from jax.experimental.pallas import tpu as pltpu
from jax.experimental.pallas import tpu_sc as plsc

VOCAB = 100000
DIM = 128
BATCH = 1024
CTX = 50

NC = 2
NS = 16
L = 16
NW = NC * NS
ROWS_PER_W = BATCH // NW


def _sc_gather_sum(x, emb_table):
    mesh = plsc.VectorSubcoreMesh(core_axis_name="c", subcore_axis_name="s")

    def _accum(rows_v, acc_v, r):
        for c in range(DIM // L):
            sl = pl.ds(c * L, L)
            vals = [rows_v[rr, sl] for rr in range(CTX)]
            while len(vals) > 1:
                nxt = [a + b for a, b in zip(vals[::2], vals[1::2])]
                if len(vals) % 2:
                    nxt.append(vals[-1])
                vals = nxt
            acc_v[r, sl] = vals[0]

    @functools.partial(
        pl.kernel,
        out_type=jax.ShapeDtypeStruct((BATCH, DIM), jnp.float32),
        mesh=mesh,
        scratch_types=[
            pltpu.VMEM((ROWS_PER_W, CTX), jnp.int32),
            pltpu.VMEM((CTX, DIM), jnp.float32),
            pltpu.VMEM((CTX, DIM), jnp.float32),
            pltpu.VMEM((ROWS_PER_W, DIM), jnp.float32),
            pltpu.SemaphoreType.DMA,
            pltpu.SemaphoreType.DMA,
        ],
    )
    def k(x_hbm, tbl_hbm, out_hbm, idx_v, rows_a, rows_b, acc_v, sem_a, sem_b):
        wid = lax.axis_index("s") * NC + lax.axis_index("c")
        base = wid * ROWS_PER_W
        pltpu.sync_copy(x_hbm.at[pl.ds(base, ROWS_PER_W)], idx_v)

        pltpu.async_copy(tbl_hbm.at[idx_v.at[0]], rows_a, sem_a)
        pltpu.async_copy(tbl_hbm.at[idx_v.at[1]], rows_b, sem_b)

        @pl.loop(0, ROWS_PER_W, step=2)
        def _(r):
            pltpu.make_async_copy(tbl_hbm.at[idx_v.at[r]], rows_a, sem_a).wait()
            _accum(rows_a, acc_v, r)

            @pl.when(r + 2 < ROWS_PER_W)
            def _():
                pltpu.async_copy(tbl_hbm.at[idx_v.at[r + 2]], rows_a, sem_a)

            pltpu.make_async_copy(
                tbl_hbm.at[idx_v.at[r + 1]], rows_b, sem_b
            ).wait()
            _accum(rows_b, acc_v, r + 1)

            @pl.when(r + 3 < ROWS_PER_W)
            def _():
                pltpu.async_copy(tbl_hbm.at[idx_v.at[r + 3]], rows_b, sem_b)

        pltpu.sync_copy(acc_v, out_hbm.at[pl.ds(base, ROWS_PER_W)])

    return k(x, emb_table)


VB = 4096
_GRID = (VOCAB + VB - 1) // VB


def _tc_project_t(h, W, brow):
    def mm(h_ref, w_ref, b_ref, o_ref):
        hb = h_ref[...].astype(jnp.bfloat16)
        wb = w_ref[...].astype(jnp.bfloat16)
        acc = lax.dot_general(
            wb, hb, (((1,), (1,)), ((), ())),
            preferred_element_type=jnp.float32,
        )
        o_ref[...] = acc + b_ref[...].T

    return pl.pallas_call(
        mm,
        grid=(_GRID,),
        in_specs=[
            pl.BlockSpec((BATCH, DIM), lambda j: (0, 0)),
            pl.BlockSpec((VB, DIM), lambda j: (j, 0)),
            pl.BlockSpec((1, VB), lambda j: (0, j)),
        ],
        out_specs=pl.BlockSpec((VB, BATCH), lambda j: (j, 0)),
        out_shape=jax.ShapeDtypeStruct((VOCAB, BATCH), jnp.float32),
        compiler_params=pltpu.CompilerParams(
            dimension_semantics=("arbitrary",),
        ),
    )(h, W, brow)


def kernel(x, emb_table, W, b):
    x = x.astype(jnp.int32)
    h = _sc_gather_sum(x, emb_table)
    lt = _tc_project_t(h, W, b.reshape(1, VOCAB))
    return lt.T
